# baseline (speedup 1.0000x reference)
import functools

import jax
import jax.numpy as jnp
from jax import lax
from jax.experimental import pallas as pl
from jax.experimental.pallas import tpu as pltpu
from jax.experimental.pallas import tpu_sc as plsc
B = 16384
T = 8
K = 362
D = 64
HALF = D // 2
NS = 16
BPW = B // NS
CH = 256
L = 16
UNROLL = 2


def _chunk(idx_vm, tab_v, acc_v):
    @plsc.parallel_loop(0, CH // UNROLL, unroll=4)
    def step(i):
        iv = idx_vm[pl.ds(i * UNROLL * T, UNROLL * T)]
        for u in range(UNROLL):
            b = i * UNROLL + u
            off = u * T
            parts_lo = []
            parts_hi = []
            for t in range(T):
                r = iv[off + t]
                parts_lo.append(tab_v[t, r, pl.ds(0, L)])
                parts_hi.append(tab_v[t, r, pl.ds(L, L)])
            lo = ((parts_lo[0] + parts_lo[1]) + (parts_lo[2] + parts_lo[3])
                  ) + ((parts_lo[4] + parts_lo[5]) + (parts_lo[6] + parts_lo[7]))
            hi = ((parts_hi[0] + parts_hi[1]) + (parts_hi[2] + parts_hi[3])
                  ) + ((parts_hi[4] + parts_hi[5]) + (parts_hi[6] + parts_hi[7]))
            acc_v[b, pl.ds(0, L)] = lo
            acc_v[b, pl.ds(L, L)] = hi
    return


def _chunk_old(idx_vm, tab_v, acc_v):
    @plsc.parallel_loop(0, CH // UNROLL, unroll=2)
    def step(i):
        ivs = [
            idx_vm[pl.ds((i * UNROLL + 2 * j) * T, 2 * T)]
            for j in range(UNROLL // 2)
        ]
        for u in range(UNROLL):
            b = i * UNROLL + u
            iv = ivs[u // 2]
            off = (u % 2) * T
            parts_lo = []
            parts_hi = []
            for t in range(T):
                r = iv[off + t]
                parts_lo.append(tab_v[t, r, pl.ds(0, L)])
                parts_hi.append(tab_v[t, r, pl.ds(L, L)])
            lo = ((parts_lo[0] + parts_lo[1]) + (parts_lo[2] + parts_lo[3])
                  ) + ((parts_lo[4] + parts_lo[5]) + (parts_lo[6] + parts_lo[7]))
            hi = ((parts_hi[0] + parts_hi[1]) + (parts_hi[2] + parts_hi[3])
                  ) + ((parts_hi[4] + parts_hi[5]) + (parts_hi[6] + parts_hi[7]))
            acc_v[b, pl.ds(0, L)] = lo
            acc_v[b, pl.ds(L, L)] = hi


def _body(lm_hbm, enc_hbm, out_hbm, tab_v, idx0, idx1, acc0, acc1,
          sem_t, sem_i0, sem_i1, sem_o0, sem_o1):
    c = lax.axis_index("c")
    s = lax.axis_index("s")
    n_chunks = BPW // CH
    idx_b = [idx0, idx1]
    acc_b = [acc0, acc1]
    sem_i = [sem_i0, sem_i1]
    sem_o = [sem_o0, sem_o1]

    def base(ch):
        return s * BPW + ch * CH

    cp_t = pltpu.async_copy(enc_hbm.at[c], tab_v, sem_t)
    cp_i = {}
    cp_o = {}
    for ch in range(min(2, n_chunks)):
        cp_i[ch] = pltpu.async_copy(
            lm_hbm.at[pl.ds(base(ch) * T, CH * T)], idx_b[ch % 2],
            sem_i[ch % 2])
    cp_t.wait()
    for ch in range(n_chunks):
        cp_i[ch].wait()
        if ch >= 2:
            cp_o[ch - 2].wait()
        _chunk(idx_b[ch % 2], tab_v, acc_b[ch % 2])
        cp_o[ch] = pltpu.async_copy(
            acc_b[ch % 2],
            out_hbm.at[pl.ds(base(ch), CH), pl.ds(c * HALF, HALF)],
            sem_o[ch % 2])
        if ch + 2 < n_chunks:
            cp_i[ch + 2] = pltpu.async_copy(
                lm_hbm.at[pl.ds(base(ch + 2) * T, CH * T)], idx_b[ch % 2],
                sem_i[ch % 2])
    for ch in range(max(0, n_chunks - 2), n_chunks):
        cp_o[ch].wait()


@functools.partial(jax.jit, static_argnames=())
def _run(last_moves, encodings):
    mesh = plsc.VectorSubcoreMesh(core_axis_name="c", subcore_axis_name="s")
    f = functools.partial(
        pl.kernel,
        out_type=jax.ShapeDtypeStruct((B, D), jnp.float32),
        mesh=mesh,
        scratch_types=[
            pltpu.VMEM((T, K, HALF), jnp.float32),
            pltpu.VMEM((CH * T,), jnp.int32),
            pltpu.VMEM((CH * T,), jnp.int32),
            pltpu.VMEM((CH, HALF), jnp.float32),
            pltpu.VMEM((CH, HALF), jnp.float32),
            pltpu.SemaphoreType.DMA,
            pltpu.SemaphoreType.DMA,
            pltpu.SemaphoreType.DMA,
            pltpu.SemaphoreType.DMA,
            pltpu.SemaphoreType.DMA,
        ],
        compiler_params=pltpu.CompilerParams(
            use_tc_tiling_on_sc=False, needs_layout_passes=False),
    )(_body)
    enc2 = jnp.transpose(
        encodings.reshape(T, K, 2, HALF), (2, 0, 1, 3))
    return f(last_moves, enc2)


def kernel(last_moves, encodings):
    return _run(last_moves.astype(jnp.int32).reshape(-1), encodings)


# speedup vs baseline: 1.0203x; 1.0203x over previous
import functools

import jax
import jax.numpy as jnp
from jax import lax
from jax.experimental import pallas as pl
from jax.experimental.pallas import tpu as pltpu
from jax.experimental.pallas import tpu_sc as plsc
B = 16384
T = 8
K = 362
D = 64
HALF = D // 2
NS = 16
BPW = B // NS
CH = 256
L = 16
UNROLL = 2


def _chunk(idx_vm, tab_v, acc_v):
    @plsc.parallel_loop(0, CH // UNROLL, unroll=4)
    def step(i):
        iv = idx_vm[pl.ds(i * UNROLL * T, UNROLL * T)]
        for u in range(UNROLL):
            b = i * UNROLL + u
            off = u * T
            parts_lo = []
            parts_hi = []
            for t in range(T):
                r = iv[off + t]
                parts_lo.append(tab_v[t, r, pl.ds(0, L)])
                parts_hi.append(tab_v[t, r, pl.ds(L, L)])
            lo = ((parts_lo[0] + parts_lo[1]) + (parts_lo[2] + parts_lo[3])
                  ) + ((parts_lo[4] + parts_lo[5]) + (parts_lo[6] + parts_lo[7]))
            hi = ((parts_hi[0] + parts_hi[1]) + (parts_hi[2] + parts_hi[3])
                  ) + ((parts_hi[4] + parts_hi[5]) + (parts_hi[6] + parts_hi[7]))
            acc_v[b, pl.ds(0, L)] = lo
            acc_v[b, pl.ds(L, L)] = hi
    return


def _chunk_old(idx_vm, tab_v, acc_v):
    @plsc.parallel_loop(0, CH // UNROLL, unroll=2)
    def step(i):
        ivs = [
            idx_vm[pl.ds((i * UNROLL + 2 * j) * T, 2 * T)]
            for j in range(UNROLL // 2)
        ]
        for u in range(UNROLL):
            b = i * UNROLL + u
            iv = ivs[u // 2]
            off = (u % 2) * T
            parts_lo = []
            parts_hi = []
            for t in range(T):
                r = iv[off + t]
                parts_lo.append(tab_v[t, r, pl.ds(0, L)])
                parts_hi.append(tab_v[t, r, pl.ds(L, L)])
            lo = ((parts_lo[0] + parts_lo[1]) + (parts_lo[2] + parts_lo[3])
                  ) + ((parts_lo[4] + parts_lo[5]) + (parts_lo[6] + parts_lo[7]))
            hi = ((parts_hi[0] + parts_hi[1]) + (parts_hi[2] + parts_hi[3])
                  ) + ((parts_hi[4] + parts_hi[5]) + (parts_hi[6] + parts_hi[7]))
            acc_v[b, pl.ds(0, L)] = lo
            acc_v[b, pl.ds(L, L)] = hi


def _body(lm_hbm, enc_hbm, out_hbm, tab_v, idx0, idx1, acc0, acc1,
          sem_t, sem_i0, sem_i1, sem_o0, sem_o1):
    c = lax.axis_index("c")
    s = lax.axis_index("s")
    n_chunks = BPW // CH
    idx_b = [idx0, idx1]
    acc_b = [acc0, acc1]
    sem_i = [sem_i0, sem_i1]
    sem_o = [sem_o0, sem_o1]

    def base(ch):
        return s * BPW + ch * CH

    cp_t = pltpu.async_copy(enc_hbm.at[:, :, pl.ds(c * HALF, HALF)],
                            tab_v, sem_t)
    cp_i = {}
    cp_o = {}
    for ch in range(min(2, n_chunks)):
        cp_i[ch] = pltpu.async_copy(
            lm_hbm.at[pl.ds(base(ch) * T, CH * T)], idx_b[ch % 2],
            sem_i[ch % 2])
    cp_t.wait()
    for ch in range(n_chunks):
        cp_i[ch].wait()
        if ch >= 2:
            cp_o[ch - 2].wait()
        _chunk(idx_b[ch % 2], tab_v, acc_b[ch % 2])
        cp_o[ch] = pltpu.async_copy(
            acc_b[ch % 2],
            out_hbm.at[pl.ds(base(ch), CH), pl.ds(c * HALF, HALF)],
            sem_o[ch % 2])
        if ch + 2 < n_chunks:
            cp_i[ch + 2] = pltpu.async_copy(
                lm_hbm.at[pl.ds(base(ch + 2) * T, CH * T)], idx_b[ch % 2],
                sem_i[ch % 2])
    for ch in range(max(0, n_chunks - 2), n_chunks):
        cp_o[ch].wait()


@functools.partial(jax.jit, static_argnames=())
def _run(last_moves, encodings):
    mesh = plsc.VectorSubcoreMesh(core_axis_name="c", subcore_axis_name="s")
    f = functools.partial(
        pl.kernel,
        out_type=jax.ShapeDtypeStruct((B, D), jnp.float32),
        mesh=mesh,
        scratch_types=[
            pltpu.VMEM((T, K, HALF), jnp.float32),
            pltpu.VMEM((CH * T,), jnp.int32),
            pltpu.VMEM((CH * T,), jnp.int32),
            pltpu.VMEM((CH, HALF), jnp.float32),
            pltpu.VMEM((CH, HALF), jnp.float32),
            pltpu.SemaphoreType.DMA,
            pltpu.SemaphoreType.DMA,
            pltpu.SemaphoreType.DMA,
            pltpu.SemaphoreType.DMA,
            pltpu.SemaphoreType.DMA,
        ],
        compiler_params=pltpu.CompilerParams(
            use_tc_tiling_on_sc=False, needs_layout_passes=False,
            skip_device_barrier=True, disable_bounds_checks=True,
            disable_semaphore_checks=True),
    )(_body)
    return f(last_moves, encodings)


def kernel(last_moves, encodings):
    return _run(last_moves.astype(jnp.int32).reshape(-1), encodings)


# speedup vs baseline: 1.0216x; 1.0012x over previous
import functools

import jax
import jax.numpy as jnp
from jax import lax
from jax.experimental import pallas as pl
from jax.experimental.pallas import tpu as pltpu
from jax.experimental.pallas import tpu_sc as plsc
B = 16384
T = 8
K = 362
D = 64
HALF = D // 2
NS = 16
BPW = B // NS
CH = 256
L = 16
UNROLL = 2


def _chunk(idx_vm, tab_v, acc_v):
    @plsc.parallel_loop(0, CH // UNROLL, unroll=4)
    def step(i):
        iv = idx_vm[pl.ds(i * UNROLL * T, UNROLL * T)]
        for u in range(UNROLL):
            b = i * UNROLL + u
            off = u * T
            parts_lo = []
            parts_hi = []
            for t in range(T):
                r = iv[off + t]
                parts_lo.append(tab_v[t, r, pl.ds(0, L)])
                parts_hi.append(tab_v[t, r, pl.ds(L, L)])
            lo = ((parts_lo[0] + parts_lo[1]) + (parts_lo[2] + parts_lo[3])
                  ) + ((parts_lo[4] + parts_lo[5]) + (parts_lo[6] + parts_lo[7]))
            hi = ((parts_hi[0] + parts_hi[1]) + (parts_hi[2] + parts_hi[3])
                  ) + ((parts_hi[4] + parts_hi[5]) + (parts_hi[6] + parts_hi[7]))
            acc_v[b, pl.ds(0, L)] = lo
            acc_v[b, pl.ds(L, L)] = hi
    return


def _body(lm_hbm, enc_hbm, out_hbm, tab_v, idx0, idx1, acc0, acc1,
          sem_t, sem_i0, sem_i1, sem_o0, sem_o1):
    c = lax.axis_index("c")
    s = lax.axis_index("s")
    n_chunks = BPW // CH
    idx_b = [idx0, idx1]
    acc_b = [acc0, acc1]
    sem_i = [sem_i0, sem_i1]
    sem_o = [sem_o0, sem_o1]

    def base(ch):
        return s * BPW + ch * CH

    cp_t = pltpu.async_copy(enc_hbm.at[:, :, pl.ds(c * HALF, HALF)],
                            tab_v, sem_t)
    cp_i = {}
    cp_o = {}
    for ch in range(min(2, n_chunks)):
        cp_i[ch] = pltpu.async_copy(
            lm_hbm.at[pl.ds(base(ch) * T, CH * T)], idx_b[ch % 2],
            sem_i[ch % 2])
    cp_t.wait()
    for ch in range(n_chunks):
        cp_i[ch].wait()
        if ch >= 2:
            cp_o[ch - 2].wait()
        _chunk(idx_b[ch % 2], tab_v, acc_b[ch % 2])
        cp_o[ch] = pltpu.async_copy(
            acc_b[ch % 2],
            out_hbm.at[pl.ds(base(ch), CH), pl.ds(c * HALF, HALF)],
            sem_o[ch % 2])
        if ch + 2 < n_chunks:
            cp_i[ch + 2] = pltpu.async_copy(
                lm_hbm.at[pl.ds(base(ch + 2) * T, CH * T)], idx_b[ch % 2],
                sem_i[ch % 2])
    for ch in range(max(0, n_chunks - 2), n_chunks):
        cp_o[ch].wait()


@functools.partial(jax.jit, static_argnames=())
def _run(last_moves, encodings):
    mesh = plsc.VectorSubcoreMesh(core_axis_name="c", subcore_axis_name="s")
    f = functools.partial(
        pl.kernel,
        out_type=jax.ShapeDtypeStruct((B, D), jnp.float32),
        mesh=mesh,
        scratch_types=[
            pltpu.VMEM((T, K, HALF), jnp.float32),
            pltpu.VMEM((CH * T,), jnp.int32),
            pltpu.VMEM((CH * T,), jnp.int32),
            pltpu.VMEM((CH, HALF), jnp.float32),
            pltpu.VMEM((CH, HALF), jnp.float32),
            pltpu.SemaphoreType.DMA,
            pltpu.SemaphoreType.DMA,
            pltpu.SemaphoreType.DMA,
            pltpu.SemaphoreType.DMA,
            pltpu.SemaphoreType.DMA,
        ],
        compiler_params=pltpu.CompilerParams(
            use_tc_tiling_on_sc=False, needs_layout_passes=False),
    )(_body)
    return f(last_moves, encodings)


def kernel(last_moves, encodings):
    return _run(last_moves.astype(jnp.int32).reshape(-1), encodings)
